# Initial kernel scaffold; baseline (speedup 1.0000x reference)
import jax
import jax.numpy as jnp
from jax.experimental import pallas as pl


def kernel(edge_index, edge_type, dynamic_emb, emb_rel, W_neighbor, W_loop, Wih_r, Whh_r, bih_r, bhh_r, Wih_e, Whh_e, bih_e, bhh_e):
    raise NotImplementedError("write your pallas kernel here")


# speedup vs baseline: 3.2225x; 3.2225x over previous
import jax
import jax.numpy as jnp
from jax import lax
from jax.experimental import pallas as pl
from jax.experimental.pallas import tpu as pltpu
from jax.experimental.pallas import tpu_sc as plsc

_SLOPE = (1.0 / 8.0 + 1.0 / 3.0) / 2.0
_NC = 2
_NS = 16
_NW = _NC * _NS
_K = 80
_CHB = 25


def _l2n(x):
    return x / (jnp.sqrt(jnp.sum(x * x, axis=-1, keepdims=True)) + 1e-12)


def _gru(x, hid, wih, whh, bih, bhh, hdim):
    gx = lax.dot_general(x, wih, (((1,), (1,)), ((), ())),
                         preferred_element_type=jnp.float32) + bih
    gh = lax.dot_general(hid, whh, (((1,), (1,)), ((), ())),
                         preferred_element_type=jnp.float32) + bhh
    rg = jax.nn.sigmoid(gx[:, :hdim] + gh[:, :hdim])
    z = jax.nn.sigmoid(gx[:, hdim:2 * hdim] + gh[:, hdim:2 * hdim])
    n = jnp.tanh(gx[:, 2 * hdim:] + rg * gh[:, 2 * hdim:])
    return (1.0 - z) * n + z * hid


def _build_pass1(N, H, R2, E):
    K = _K
    CB = E // (_NW * K * _CHB)
    HN = N // 2
    STR = (HN // _NS) // 8 * 8
    TAIL = HN - STR * _NS
    mesh = plsc.VectorSubcoreMesh(core_axis_name="c", subcore_axis_name="s")
    out_type = (
        jax.ShapeDtypeStruct((_NC, 2, HN, H), jnp.float32),
        jax.ShapeDtypeStruct((_NC, R2, H), jnp.float32),
        jax.ShapeDtypeStruct((_NC, 2, HN + 8), jnp.float32),
        jax.ShapeDtypeStruct((_NC, R2), jnp.float32),
    )
    scratch = [
        pltpu.VMEM((_CHB, K), jnp.int32),
        pltpu.VMEM((_CHB, K), jnp.int32),
        pltpu.VMEM((_CHB, K), jnp.int32),
        pltpu.VMEM((K, H), jnp.float32),
        pltpu.VMEM((K,), jnp.float32),
        pltpu.VMEM_SHARED((HN + 8, H), jnp.float32),
        pltpu.VMEM_SHARED((R2, H), jnp.float32),
        pltpu.VMEM_SHARED((HN + 8,), jnp.float32),
        pltpu.VMEM_SHARED((R2,), jnp.float32),
        pltpu.SemaphoreType.DMA,
    ]

    def body(src_hbm, d0_hbm, d1_hbm, et_hbm, h_hbm, z2_hbm, z2r_hbm,
             z1_hbm, z1r_hbm, one_hbm,
             a_out, rs_out, deg_out, rcnt_out,
             src_v, dst_v, et_v, rows_v, ones_v,
             a_sh, rs_sh, deg_sh, rcnt_sh, sem):
        c = lax.axis_index("c")
        s = lax.axis_index("s")
        wid = c * _NS + s
        pltpu.sync_copy(one_hbm, ones_v)

        def zero_acc():
            pltpu.sync_copy(z2_hbm.at[pl.ds(s * STR, STR)],
                            a_sh.at[pl.ds(s * STR, STR)])

            @pl.when(s == _NS - 1)
            def _():
                pltpu.sync_copy(z2_hbm.at[pl.ds(STR * _NS, TAIL)],
                                a_sh.at[pl.ds(STR * _NS, TAIL)])

            @pl.when(s == 0)
            def _():
                pltpu.sync_copy(z1_hbm, deg_sh)

        def flush_acc(p):
            pltpu.sync_copy(a_sh.at[pl.ds(s * STR, STR)],
                            a_out.at[c, p, pl.ds(s * STR, STR)])

            @pl.when(s == _NS - 1)
            def _():
                pltpu.sync_copy(a_sh.at[pl.ds(STR * _NS, TAIL)],
                                a_out.at[c, p, pl.ds(STR * _NS, TAIL)])

            @pl.when(s == 0)
            def _():
                pltpu.sync_copy(deg_sh, deg_out.at[c, p])

        zero_acc()

        @pl.when(s == 0)
        def _():
            pltpu.sync_copy(z2r_hbm, rs_sh)
            pltpu.sync_copy(z1r_hbm, rcnt_sh)

        plsc.subcore_barrier()

        def block0(b, carry):
            pltpu.sync_copy(src_hbm.at[wid, b], src_v)
            pltpu.sync_copy(d0_hbm.at[wid, b], dst_v)
            pltpu.sync_copy(et_hbm.at[wid, b], et_v)

            def chunk(j, carry2):
                sidx = src_v.at[j]
                didx = dst_v.at[j]
                eidx = et_v.at[j]
                pltpu.async_copy(h_hbm.at[sidx], rows_v, sem).wait()
                pltpu.sync_copy(rows_v, a_sh.at[didx], add=True)
                pltpu.sync_copy(rows_v, rs_sh.at[eidx], add=True)
                pltpu.sync_copy(ones_v, deg_sh.at[didx], add=True)
                pltpu.sync_copy(ones_v, rcnt_sh.at[eidx], add=True)
                return carry2

            lax.fori_loop(0, _CHB, chunk, 0)
            return carry

        lax.fori_loop(0, CB, block0, 0)
        plsc.subcore_barrier()

        flush_acc(0)

        @pl.when(s == 0)
        def _():
            pltpu.sync_copy(rs_sh, rs_out.at[c])
            pltpu.sync_copy(rcnt_sh, rcnt_out.at[c])

        zero_acc()
        plsc.subcore_barrier()

        def block1(b, carry):
            pltpu.sync_copy(src_hbm.at[wid, b], src_v)
            pltpu.sync_copy(d1_hbm.at[wid, b], dst_v)

            def chunk(j, carry2):
                sidx = src_v.at[j]
                didx = dst_v.at[j]
                pltpu.async_copy(h_hbm.at[sidx], rows_v, sem).wait()
                pltpu.sync_copy(rows_v, a_sh.at[didx], add=True)
                pltpu.sync_copy(ones_v, deg_sh.at[didx], add=True)
                return carry2

            lax.fori_loop(0, _CHB, chunk, 0)
            return carry

        lax.fori_loop(0, CB, block1, 0)
        plsc.subcore_barrier()
        flush_acc(1)

    return pl.kernel(body, out_type=out_type, mesh=mesh,
                     scratch_types=scratch)


def _build_pass2(N, H, R2, E):
    K = _K
    CB = E // (_NW * K * _CHB)
    HN = N // 2
    STR = (HN // _NS) // 8 * 8
    TAIL = HN - STR * _NS
    mesh = plsc.VectorSubcoreMesh(core_axis_name="c", subcore_axis_name="s")
    out_type = jax.ShapeDtypeStruct((_NC, 2, HN, H), jnp.float32)
    scratch = [
        pltpu.VMEM((_CHB, K), jnp.int32),
        pltpu.VMEM((_CHB, K), jnp.int32),
        pltpu.VMEM((K, H), jnp.float32),
        pltpu.VMEM_SHARED((HN + 8, H), jnp.float32),
        pltpu.SemaphoreType.DMA,
    ]

    def body(d0_hbm, d1_hbm, et_hbm, r_hbm, z2_hbm,
             b_out, dst_v, et_v, rows_v, b_sh, sem):
        c = lax.axis_index("c")
        s = lax.axis_index("s")
        wid = c * _NS + s

        def phase(d_hbm, p):
            pltpu.sync_copy(z2_hbm.at[pl.ds(s * STR, STR)],
                            b_sh.at[pl.ds(s * STR, STR)])

            @pl.when(s == _NS - 1)
            def _():
                pltpu.sync_copy(z2_hbm.at[pl.ds(STR * _NS, TAIL)],
                                b_sh.at[pl.ds(STR * _NS, TAIL)])

            plsc.subcore_barrier()

            def block(b, carry):
                pltpu.sync_copy(d_hbm.at[wid, b], dst_v)
                pltpu.sync_copy(et_hbm.at[wid, b], et_v)

                def chunk(j, carry2):
                    pltpu.async_copy(r_hbm.at[et_v.at[j]], rows_v,
                                     sem).wait()
                    pltpu.sync_copy(rows_v, b_sh.at[dst_v.at[j]], add=True)
                    return carry2

                lax.fori_loop(0, _CHB, chunk, 0)
                return carry

            lax.fori_loop(0, CB, block, 0)
            plsc.subcore_barrier()
            pltpu.sync_copy(b_sh.at[pl.ds(s * STR, STR)],
                            b_out.at[c, p, pl.ds(s * STR, STR)])

            @pl.when(s == _NS - 1)
            def _():
                pltpu.sync_copy(b_sh.at[pl.ds(STR * _NS, TAIL)],
                                b_out.at[c, p, pl.ds(STR * _NS, TAIL)])

        phase(d0_hbm, 0)
        plsc.subcore_barrier()
        phase(d1_hbm, 1)

    return pl.kernel(body, out_type=out_type, mesh=mesh,
                     scratch_types=scratch)


def _prep(demb):
    N, H = demb.shape
    RB = 2000

    def body(d_ref, h_ref):
        h_ref[...] = _l2n(d_ref[...])

    return pl.pallas_call(
        body,
        grid=(N // RB,),
        in_specs=[pl.BlockSpec((RB, H), lambda i: (i, 0))],
        out_specs=pl.BlockSpec((RB, H), lambda i: (i, 0)),
        out_shape=jax.ShapeDtypeStruct((N, H), jnp.float32),
    )(demb)


def _tc_rel(emb_rel, r_prev, rs_p, rcnt_p, wih, whh, bih, bhh):
    R2, H = emb_rel.shape

    def body(emb_ref, r_ref, rs_ref, rcnt_ref, wih_ref, whh_ref,
             bih_ref, bhh_ref, rout_ref):
        rs = rs_ref[0] + rs_ref[1]
        cnt = jnp.maximum(jnp.sum(rcnt_ref[...], axis=0), 1.0)
        x = jnp.concatenate([emb_ref[...], rs / cnt], axis=1)
        rout_ref[...] = _gru(x, r_ref[...], wih_ref[...], whh_ref[...],
                             bih_ref[...], bhh_ref[...], H)

    return pl.pallas_call(
        body,
        out_shape=jax.ShapeDtypeStruct((R2, H), jnp.float32),
    )(emb_rel, r_prev, rs_p, rcnt_p, wih, whh, bih, bhh)


def _tc_ent(a_p, b_p, deg_p, h, wn, wl, wih, whh, bih, bhh):
    N, H = h.shape
    RB = 2000

    def body(a_ref, b_ref, deg_ref, h_ref, wn_ref, wl_ref, wih_ref,
             whh_ref, bih_ref, bhh_ref, hout_ref):
        acc = a_ref[0] + a_ref[1] - b_ref[0] - b_ref[1]
        deg = jnp.maximum(jnp.sum(deg_ref[...], axis=0), 1.0)
        agg = jnp.dot(acc, wn_ref[...],
                      preferred_element_type=jnp.float32) / deg
        cur = agg + jnp.dot(h_ref[...], wl_ref[...],
                            preferred_element_type=jnp.float32)
        cur = jnp.where(cur >= 0, cur, _SLOPE * cur)
        cur = _l2n(cur)
        hn = _gru(cur, h_ref[...], wih_ref[...], whh_ref[...],
                  bih_ref[...], bhh_ref[...], H)
        hout_ref[...] = _l2n(hn)

    c0 = lambda i: (0, i, 0)
    return pl.pallas_call(
        body,
        grid=(N // RB,),
        in_specs=[pl.BlockSpec((2, RB, H), c0),
                  pl.BlockSpec((2, RB, H), c0),
                  pl.BlockSpec((2, RB, 1), c0),
                  pl.BlockSpec((RB, H), lambda i: (i, 0)),
                  pl.BlockSpec((H, H), lambda i: (0, 0)),
                  pl.BlockSpec((H, H), lambda i: (0, 0)),
                  pl.BlockSpec((3 * H, H), lambda i: (0, 0)),
                  pl.BlockSpec((3 * H, H), lambda i: (0, 0)),
                  pl.BlockSpec((1, 3 * H), lambda i: (0, 0)),
                  pl.BlockSpec((1, 3 * H), lambda i: (0, 0))],
        out_specs=pl.BlockSpec((RB, H), lambda i: (i, 0)),
        out_shape=jax.ShapeDtypeStruct((N, H), jnp.float32),
    )(a_p, b_p, deg_p, h, wn, wl, wih, whh, bih, bhh)


def kernel(edge_index, edge_type, dynamic_emb, emb_rel, W_neighbor, W_loop,
           Wih_r, Whh_r, bih_r, bhh_r, Wih_e, Whh_e, bih_e, bhh_e):
    N, H = dynamic_emb.shape
    R2 = emb_rel.shape[0]
    T, E = edge_type.shape
    f32 = jnp.float32
    CB = E // (_NW * _K * _CHB)
    HN = N // 2

    z2 = jnp.zeros((HN, H), f32)
    z2r = jnp.zeros((R2, H), f32)
    z1 = jnp.zeros((HN + 8,), f32)
    z1r = jnp.zeros((R2,), f32)
    one = jnp.ones((_K,), f32)
    bih_r2 = bih_r.reshape(1, -1)
    bhh_r2 = bhh_r.reshape(1, -1)
    bih_e2 = bih_e.reshape(1, -1)
    bhh_e2 = bhh_e.reshape(1, -1)

    pass1 = _build_pass1(N, H, R2, E)
    pass2 = _build_pass2(N, H, R2, E)

    h = _prep(dynamic_emb)
    r = emb_rel
    for t in range(T):
        src = edge_index[t, 0].astype(jnp.int32).reshape(_NW, CB, _CHB, _K)
        dst = edge_index[t, 1].astype(jnp.int32)
        d0 = jnp.where(dst < HN, dst, HN).reshape(_NW, CB, _CHB, _K)
        d1 = jnp.where(dst >= HN, dst - HN, HN).reshape(_NW, CB, _CHB, _K)
        et = edge_type[t].astype(jnp.int32).reshape(_NW, CB, _CHB, _K)
        a_p, rs_p, deg_p, rcnt_p = pass1(src, d0, d1, et, h, z2, z2r, z1,
                                         z1r, one)
        r = _tc_rel(emb_rel, r, rs_p, rcnt_p.reshape(_NC, R2, 1),
                    Wih_r, Whh_r, bih_r2, bhh_r2)
        b_p = pass2(d0, d1, et, r, z2)
        h = _tc_ent(a_p.reshape(_NC, N, H), b_p.reshape(_NC, N, H),
                    deg_p[:, :, :HN].reshape(_NC, N, 1), h, W_neighbor,
                    W_loop, Wih_e, Whh_e, bih_e2, bhh_e2)
    return h


# speedup vs baseline: 4.0709x; 1.2633x over previous
import jax
import jax.numpy as jnp
from jax import lax
from jax.experimental import pallas as pl
from jax.experimental.pallas import tpu as pltpu
from jax.experimental.pallas import tpu_sc as plsc

_SLOPE = (1.0 / 8.0 + 1.0 / 3.0) / 2.0
_NC = 2
_NS = 16
_NW = _NC * _NS
_K = 80
_CHB = 25


def _l2n(x):
    return x / (jnp.sqrt(jnp.sum(x * x, axis=-1, keepdims=True)) + 1e-12)


def _gru(x, hid, wih, whh, bih, bhh, hdim):
    gx = lax.dot_general(x, wih, (((1,), (1,)), ((), ())),
                         preferred_element_type=jnp.float32) + bih
    gh = lax.dot_general(hid, whh, (((1,), (1,)), ((), ())),
                         preferred_element_type=jnp.float32) + bhh
    rg = jax.nn.sigmoid(gx[:, :hdim] + gh[:, :hdim])
    z = jax.nn.sigmoid(gx[:, hdim:2 * hdim] + gh[:, hdim:2 * hdim])
    n = jnp.tanh(gx[:, 2 * hdim:] + rg * gh[:, 2 * hdim:])
    return (1.0 - z) * n + z * hid


def _build_pass1(N, H, R2, E):
    K = _K
    CB = E // (_NW * K * _CHB)
    HN = N // 2
    STR = (HN // _NS) // 8 * 8
    TAIL = HN - STR * _NS
    mesh = plsc.VectorSubcoreMesh(core_axis_name="c", subcore_axis_name="s")
    out_type = (
        jax.ShapeDtypeStruct((_NC, 2, HN, H), jnp.float32),
        jax.ShapeDtypeStruct((_NC, R2, H), jnp.float32),
        jax.ShapeDtypeStruct((_NC, 2, HN + 8), jnp.float32),
        jax.ShapeDtypeStruct((_NC, R2), jnp.float32),
    )
    scratch = [
        pltpu.VMEM((_CHB, K), jnp.int32),
        pltpu.VMEM((_CHB, K), jnp.int32),
        pltpu.VMEM((_CHB, K), jnp.int32),
        pltpu.VMEM((K, H), jnp.float32),
        pltpu.VMEM((K, H), jnp.float32),
        pltpu.VMEM((K,), jnp.float32),
        pltpu.VMEM_SHARED((HN + 8, H), jnp.float32),
        pltpu.VMEM_SHARED((R2, H), jnp.float32),
        pltpu.VMEM_SHARED((HN + 8,), jnp.float32),
        pltpu.VMEM_SHARED((R2,), jnp.float32),
        pltpu.SemaphoreType.DMA,
        pltpu.SemaphoreType.DMA,
    ]

    def body(src_hbm, d0_hbm, d1_hbm, et_hbm, h_hbm, z2_hbm, z2r_hbm,
             z1_hbm, z1r_hbm, one_hbm,
             a_out, rs_out, deg_out, rcnt_out,
             src_v, dst_v, et_v, rows_a, rows_b, ones_v,
             a_sh, rs_sh, deg_sh, rcnt_sh, sem_a, sem_b):
        c = lax.axis_index("c")
        s = lax.axis_index("s")
        wid = c * _NS + s
        pltpu.sync_copy(one_hbm, ones_v)

        def zero_acc():
            pltpu.sync_copy(z2_hbm.at[pl.ds(s * STR, STR)],
                            a_sh.at[pl.ds(s * STR, STR)])

            @pl.when(s == _NS - 1)
            def _():
                pltpu.sync_copy(z2_hbm.at[pl.ds(STR * _NS, TAIL)],
                                a_sh.at[pl.ds(STR * _NS, TAIL)])

            @pl.when(s == 0)
            def _():
                pltpu.sync_copy(z1_hbm, deg_sh)

        def flush_acc(p):
            pltpu.sync_copy(a_sh.at[pl.ds(s * STR, STR)],
                            a_out.at[c, p, pl.ds(s * STR, STR)])

            @pl.when(s == _NS - 1)
            def _():
                pltpu.sync_copy(a_sh.at[pl.ds(STR * _NS, TAIL)],
                                a_out.at[c, p, pl.ds(STR * _NS, TAIL)])

            @pl.when(s == 0)
            def _():
                pltpu.sync_copy(deg_sh, deg_out.at[c, p])

        zero_acc()

        @pl.when(s == 0)
        def _():
            pltpu.sync_copy(z2r_hbm, rs_sh)
            pltpu.sync_copy(z1r_hbm, rcnt_sh)

        plsc.subcore_barrier()

        def block0(b, carry):
            pltpu.sync_copy(src_hbm.at[wid, b], src_v)
            pltpu.sync_copy(d0_hbm.at[wid, b], dst_v)
            pltpu.sync_copy(et_hbm.at[wid, b], et_v)
            pltpu.async_copy(h_hbm.at[src_v.at[0]], rows_a, sem_a)

            def scat(rows, j):
                pltpu.sync_copy(rows, a_sh.at[dst_v.at[j]], add=True)
                pltpu.sync_copy(rows, rs_sh.at[et_v.at[j]], add=True)
                pltpu.sync_copy(ones_v, deg_sh.at[dst_v.at[j]], add=True)
                pltpu.sync_copy(ones_v, rcnt_sh.at[et_v.at[j]], add=True)

            def chunk(j, carry2):
                @pl.when(j % 2 == 0)
                def _():
                    pltpu.make_async_copy(h_hbm.at[src_v.at[j]], rows_a,
                                          sem_a).wait()

                    @pl.when(j < _CHB - 1)
                    def _():
                        pltpu.async_copy(h_hbm.at[src_v.at[j + 1]],
                                         rows_b, sem_b)

                    scat(rows_a, j)

                @pl.when(j % 2 == 1)
                def _():
                    pltpu.make_async_copy(h_hbm.at[src_v.at[j]], rows_b,
                                          sem_b).wait()

                    @pl.when(j < _CHB - 1)
                    def _():
                        pltpu.async_copy(h_hbm.at[src_v.at[j + 1]],
                                         rows_a, sem_a)

                    scat(rows_b, j)
                return carry2

            lax.fori_loop(0, _CHB, chunk, 0)
            return carry

        lax.fori_loop(0, CB, block0, 0)
        plsc.subcore_barrier()

        flush_acc(0)

        @pl.when(s == 0)
        def _():
            pltpu.sync_copy(rs_sh, rs_out.at[c])
            pltpu.sync_copy(rcnt_sh, rcnt_out.at[c])

        zero_acc()
        plsc.subcore_barrier()

        def block1(b, carry):
            pltpu.sync_copy(src_hbm.at[wid, b], src_v)
            pltpu.sync_copy(d1_hbm.at[wid, b], dst_v)
            pltpu.async_copy(h_hbm.at[src_v.at[0]], rows_a, sem_a)

            def scat(rows, j):
                pltpu.sync_copy(rows, a_sh.at[dst_v.at[j]], add=True)
                pltpu.sync_copy(ones_v, deg_sh.at[dst_v.at[j]], add=True)

            def chunk(j, carry2):
                @pl.when(j % 2 == 0)
                def _():
                    pltpu.make_async_copy(h_hbm.at[src_v.at[j]], rows_a,
                                          sem_a).wait()

                    @pl.when(j < _CHB - 1)
                    def _():
                        pltpu.async_copy(h_hbm.at[src_v.at[j + 1]],
                                         rows_b, sem_b)

                    scat(rows_a, j)

                @pl.when(j % 2 == 1)
                def _():
                    pltpu.make_async_copy(h_hbm.at[src_v.at[j]], rows_b,
                                          sem_b).wait()

                    @pl.when(j < _CHB - 1)
                    def _():
                        pltpu.async_copy(h_hbm.at[src_v.at[j + 1]],
                                         rows_a, sem_a)

                    scat(rows_b, j)
                return carry2

            lax.fori_loop(0, _CHB, chunk, 0)
            return carry

        lax.fori_loop(0, CB, block1, 0)
        plsc.subcore_barrier()
        flush_acc(1)

    return pl.kernel(body, out_type=out_type, mesh=mesh,
                     scratch_types=scratch)


def _build_pass2(N, H, R2, E):
    K = _K
    CB = E // (_NW * K * _CHB)
    HN = N // 2
    STR = (HN // _NS) // 8 * 8
    TAIL = HN - STR * _NS
    mesh = plsc.VectorSubcoreMesh(core_axis_name="c", subcore_axis_name="s")
    out_type = jax.ShapeDtypeStruct((_NC, 2, HN, H), jnp.float32)
    scratch = [
        pltpu.VMEM((_CHB, K), jnp.int32),
        pltpu.VMEM((_CHB, K), jnp.int32),
        pltpu.VMEM((K, H), jnp.float32),
        pltpu.VMEM((K, H), jnp.float32),
        pltpu.VMEM_SHARED((HN + 8, H), jnp.float32),
        pltpu.SemaphoreType.DMA,
        pltpu.SemaphoreType.DMA,
    ]

    def body(d0_hbm, d1_hbm, et_hbm, r_hbm, z2_hbm,
             b_out, dst_v, et_v, rows_a, rows_b, b_sh, sem_a, sem_b):
        c = lax.axis_index("c")
        s = lax.axis_index("s")
        wid = c * _NS + s

        def phase(d_hbm, p):
            pltpu.sync_copy(z2_hbm.at[pl.ds(s * STR, STR)],
                            b_sh.at[pl.ds(s * STR, STR)])

            @pl.when(s == _NS - 1)
            def _():
                pltpu.sync_copy(z2_hbm.at[pl.ds(STR * _NS, TAIL)],
                                b_sh.at[pl.ds(STR * _NS, TAIL)])

            plsc.subcore_barrier()

            def block(b, carry):
                pltpu.sync_copy(d_hbm.at[wid, b], dst_v)
                pltpu.sync_copy(et_hbm.at[wid, b], et_v)
                pltpu.async_copy(r_hbm.at[et_v.at[0]], rows_a, sem_a)

                def chunk(j, carry2):
                    @pl.when(j % 2 == 0)
                    def _():
                        pltpu.make_async_copy(r_hbm.at[et_v.at[j]],
                                              rows_a, sem_a).wait()

                        @pl.when(j < _CHB - 1)
                        def _():
                            pltpu.async_copy(r_hbm.at[et_v.at[j + 1]],
                                             rows_b, sem_b)

                        pltpu.sync_copy(rows_a, b_sh.at[dst_v.at[j]],
                                        add=True)

                    @pl.when(j % 2 == 1)
                    def _():
                        pltpu.make_async_copy(r_hbm.at[et_v.at[j]],
                                              rows_b, sem_b).wait()

                        @pl.when(j < _CHB - 1)
                        def _():
                            pltpu.async_copy(r_hbm.at[et_v.at[j + 1]],
                                             rows_a, sem_a)

                        pltpu.sync_copy(rows_b, b_sh.at[dst_v.at[j]],
                                        add=True)
                    return carry2

                lax.fori_loop(0, _CHB, chunk, 0)
                return carry

            lax.fori_loop(0, CB, block, 0)
            plsc.subcore_barrier()
            pltpu.sync_copy(b_sh.at[pl.ds(s * STR, STR)],
                            b_out.at[c, p, pl.ds(s * STR, STR)])

            @pl.when(s == _NS - 1)
            def _():
                pltpu.sync_copy(b_sh.at[pl.ds(STR * _NS, TAIL)],
                                b_out.at[c, p, pl.ds(STR * _NS, TAIL)])

        phase(d0_hbm, 0)
        plsc.subcore_barrier()
        phase(d1_hbm, 1)

    return pl.kernel(body, out_type=out_type, mesh=mesh,
                     scratch_types=scratch)


def _prep(demb):
    N, H = demb.shape
    RB = 2000

    def body(d_ref, h_ref):
        h_ref[...] = _l2n(d_ref[...])

    return pl.pallas_call(
        body,
        grid=(N // RB,),
        in_specs=[pl.BlockSpec((RB, H), lambda i: (i, 0))],
        out_specs=pl.BlockSpec((RB, H), lambda i: (i, 0)),
        out_shape=jax.ShapeDtypeStruct((N, H), jnp.float32),
    )(demb)


def _tc_rel(emb_rel, r_prev, rs_p, rcnt_p, wih, whh, bih, bhh):
    R2, H = emb_rel.shape

    def body(emb_ref, r_ref, rs_ref, rcnt_ref, wih_ref, whh_ref,
             bih_ref, bhh_ref, rout_ref):
        rs = rs_ref[0] + rs_ref[1]
        cnt = jnp.maximum(jnp.sum(rcnt_ref[...], axis=0), 1.0)
        x = jnp.concatenate([emb_ref[...], rs / cnt], axis=1)
        rout_ref[...] = _gru(x, r_ref[...], wih_ref[...], whh_ref[...],
                             bih_ref[...], bhh_ref[...], H)

    return pl.pallas_call(
        body,
        out_shape=jax.ShapeDtypeStruct((R2, H), jnp.float32),
    )(emb_rel, r_prev, rs_p, rcnt_p, wih, whh, bih, bhh)


def _tc_ent(a_p, b_p, deg_p, h, wn, wl, wih, whh, bih, bhh):
    N, H = h.shape
    RB = 2000

    def body(a_ref, b_ref, deg_ref, h_ref, wn_ref, wl_ref, wih_ref,
             whh_ref, bih_ref, bhh_ref, hout_ref):
        acc = a_ref[0] + a_ref[1] - b_ref[0] - b_ref[1]
        deg = jnp.maximum(jnp.sum(deg_ref[...], axis=0), 1.0)
        agg = jnp.dot(acc, wn_ref[...],
                      preferred_element_type=jnp.float32) / deg
        cur = agg + jnp.dot(h_ref[...], wl_ref[...],
                            preferred_element_type=jnp.float32)
        cur = jnp.where(cur >= 0, cur, _SLOPE * cur)
        cur = _l2n(cur)
        hn = _gru(cur, h_ref[...], wih_ref[...], whh_ref[...],
                  bih_ref[...], bhh_ref[...], H)
        hout_ref[...] = _l2n(hn)

    c0 = lambda i: (0, i, 0)
    return pl.pallas_call(
        body,
        grid=(N // RB,),
        in_specs=[pl.BlockSpec((2, RB, H), c0),
                  pl.BlockSpec((2, RB, H), c0),
                  pl.BlockSpec((2, RB, 1), c0),
                  pl.BlockSpec((RB, H), lambda i: (i, 0)),
                  pl.BlockSpec((H, H), lambda i: (0, 0)),
                  pl.BlockSpec((H, H), lambda i: (0, 0)),
                  pl.BlockSpec((3 * H, H), lambda i: (0, 0)),
                  pl.BlockSpec((3 * H, H), lambda i: (0, 0)),
                  pl.BlockSpec((1, 3 * H), lambda i: (0, 0)),
                  pl.BlockSpec((1, 3 * H), lambda i: (0, 0))],
        out_specs=pl.BlockSpec((RB, H), lambda i: (i, 0)),
        out_shape=jax.ShapeDtypeStruct((N, H), jnp.float32),
    )(a_p, b_p, deg_p, h, wn, wl, wih, whh, bih, bhh)


def kernel(edge_index, edge_type, dynamic_emb, emb_rel, W_neighbor, W_loop,
           Wih_r, Whh_r, bih_r, bhh_r, Wih_e, Whh_e, bih_e, bhh_e):
    N, H = dynamic_emb.shape
    R2 = emb_rel.shape[0]
    T, E = edge_type.shape
    f32 = jnp.float32
    CB = E // (_NW * _K * _CHB)
    HN = N // 2

    z2 = jnp.zeros((HN, H), f32)
    z2r = jnp.zeros((R2, H), f32)
    z1 = jnp.zeros((HN + 8,), f32)
    z1r = jnp.zeros((R2,), f32)
    one = jnp.ones((_K,), f32)
    bih_r2 = bih_r.reshape(1, -1)
    bhh_r2 = bhh_r.reshape(1, -1)
    bih_e2 = bih_e.reshape(1, -1)
    bhh_e2 = bhh_e.reshape(1, -1)

    pass1 = _build_pass1(N, H, R2, E)
    pass2 = _build_pass2(N, H, R2, E)

    h = _prep(dynamic_emb)
    r = emb_rel
    for t in range(T):
        src = edge_index[t, 0].astype(jnp.int32).reshape(_NW, CB, _CHB, _K)
        dst = edge_index[t, 1].astype(jnp.int32)
        d0 = jnp.where(dst < HN, dst, HN).reshape(_NW, CB, _CHB, _K)
        d1 = jnp.where(dst >= HN, dst - HN, HN).reshape(_NW, CB, _CHB, _K)
        et = edge_type[t].astype(jnp.int32).reshape(_NW, CB, _CHB, _K)
        a_p, rs_p, deg_p, rcnt_p = pass1(src, d0, d1, et, h, z2, z2r, z1,
                                         z1r, one)
        r = _tc_rel(emb_rel, r, rs_p, rcnt_p.reshape(_NC, R2, 1),
                    Wih_r, Whh_r, bih_r2, bhh_r2)
        b_p = pass2(d0, d1, et, r, z2)
        h = _tc_ent(a_p.reshape(_NC, N, H), b_p.reshape(_NC, N, H),
                    deg_p[:, :, :HN].reshape(_NC, N, 1), h, W_neighbor,
                    W_loop, Wih_e, Whh_e, bih_e2, bhh_e2)
    return h


# speedup vs baseline: 4.1428x; 1.0177x over previous
import jax
import jax.numpy as jnp
from jax import lax
from jax.experimental import pallas as pl
from jax.experimental.pallas import tpu as pltpu
from jax.experimental.pallas import tpu_sc as plsc

_SLOPE = (1.0 / 8.0 + 1.0 / 3.0) / 2.0
_NC = 2
_NS = 16
_NW = _NC * _NS
_K = 100
_CHB = 20


def _l2n(x):
    return x / (jnp.sqrt(jnp.sum(x * x, axis=-1, keepdims=True)) + 1e-12)


def _gru(x, hid, wih, whh, bih, bhh, hdim):
    gx = lax.dot_general(x, wih, (((1,), (1,)), ((), ())),
                         preferred_element_type=jnp.float32) + bih
    gh = lax.dot_general(hid, whh, (((1,), (1,)), ((), ())),
                         preferred_element_type=jnp.float32) + bhh
    rg = jax.nn.sigmoid(gx[:, :hdim] + gh[:, :hdim])
    z = jax.nn.sigmoid(gx[:, hdim:2 * hdim] + gh[:, hdim:2 * hdim])
    n = jnp.tanh(gx[:, 2 * hdim:] + rg * gh[:, 2 * hdim:])
    return (1.0 - z) * n + z * hid


def _build_pass1(N, H, R2, E):
    K = _K
    CB = E // (_NW * K * _CHB)
    HN = N // 2
    STR = (HN // _NS) // 8 * 8
    TAIL = HN - STR * _NS
    mesh = plsc.VectorSubcoreMesh(core_axis_name="c", subcore_axis_name="s")
    out_type = (
        jax.ShapeDtypeStruct((_NC, 2, HN, H), jnp.float32),
        jax.ShapeDtypeStruct((_NC, R2, H), jnp.float32),
        jax.ShapeDtypeStruct((_NC, 2, HN + 8), jnp.float32),
        jax.ShapeDtypeStruct((_NC, R2), jnp.float32),
    )
    scratch = [
        pltpu.VMEM((_CHB, K), jnp.int32),
        pltpu.VMEM((_CHB, K), jnp.int32),
        pltpu.VMEM((_CHB, K), jnp.int32),
        pltpu.VMEM((K, H), jnp.float32),
        pltpu.VMEM((K, H), jnp.float32),
        pltpu.VMEM((K,), jnp.float32),
        pltpu.VMEM_SHARED((HN + 8, H), jnp.float32),
        pltpu.VMEM_SHARED((R2, H), jnp.float32),
        pltpu.VMEM_SHARED((HN + 8,), jnp.float32),
        pltpu.VMEM_SHARED((R2,), jnp.float32),
        pltpu.SemaphoreType.DMA,
        pltpu.SemaphoreType.DMA,
    ]

    def body(src_hbm, d0_hbm, d1_hbm, et_hbm, h_hbm, z2_hbm, z2r_hbm,
             z1_hbm, z1r_hbm, one_hbm,
             a_out, rs_out, deg_out, rcnt_out,
             src_v, dst_v, et_v, rows_a, rows_b, ones_v,
             a_sh, rs_sh, deg_sh, rcnt_sh, sem_a, sem_b):
        c = lax.axis_index("c")
        s = lax.axis_index("s")
        wid = c * _NS + s
        pltpu.sync_copy(one_hbm, ones_v)

        def zero_acc():
            pltpu.sync_copy(z2_hbm.at[pl.ds(s * STR, STR)],
                            a_sh.at[pl.ds(s * STR, STR)])

            @pl.when(s == _NS - 1)
            def _():
                pltpu.sync_copy(z2_hbm.at[pl.ds(STR * _NS, TAIL)],
                                a_sh.at[pl.ds(STR * _NS, TAIL)])

            @pl.when(s == 0)
            def _():
                pltpu.sync_copy(z1_hbm, deg_sh)

        def flush_acc(p):
            pltpu.sync_copy(a_sh.at[pl.ds(s * STR, STR)],
                            a_out.at[c, p, pl.ds(s * STR, STR)])

            @pl.when(s == _NS - 1)
            def _():
                pltpu.sync_copy(a_sh.at[pl.ds(STR * _NS, TAIL)],
                                a_out.at[c, p, pl.ds(STR * _NS, TAIL)])

            @pl.when(s == 0)
            def _():
                pltpu.sync_copy(deg_sh, deg_out.at[c, p])

        zero_acc()

        @pl.when(s == 0)
        def _():
            pltpu.sync_copy(z2r_hbm, rs_sh)
            pltpu.sync_copy(z1r_hbm, rcnt_sh)

        plsc.subcore_barrier()

        def block0(b, carry):
            pltpu.sync_copy(src_hbm.at[wid, b], src_v)
            pltpu.sync_copy(d0_hbm.at[wid, b], dst_v)
            pltpu.sync_copy(et_hbm.at[wid, b], et_v)
            pltpu.async_copy(h_hbm.at[src_v.at[0]], rows_a, sem_a)

            def scat(rows, j):
                pltpu.sync_copy(rows, a_sh.at[dst_v.at[j]], add=True)
                pltpu.sync_copy(rows, rs_sh.at[et_v.at[j]], add=True)
                pltpu.sync_copy(ones_v, deg_sh.at[dst_v.at[j]], add=True)
                pltpu.sync_copy(ones_v, rcnt_sh.at[et_v.at[j]], add=True)

            def chunk(j, carry2):
                @pl.when(j % 2 == 0)
                def _():
                    pltpu.make_async_copy(h_hbm.at[src_v.at[j]], rows_a,
                                          sem_a).wait()

                    @pl.when(j < _CHB - 1)
                    def _():
                        pltpu.async_copy(h_hbm.at[src_v.at[j + 1]],
                                         rows_b, sem_b)

                    scat(rows_a, j)

                @pl.when(j % 2 == 1)
                def _():
                    pltpu.make_async_copy(h_hbm.at[src_v.at[j]], rows_b,
                                          sem_b).wait()

                    @pl.when(j < _CHB - 1)
                    def _():
                        pltpu.async_copy(h_hbm.at[src_v.at[j + 1]],
                                         rows_a, sem_a)

                    scat(rows_b, j)
                return carry2

            lax.fori_loop(0, _CHB, chunk, 0)
            return carry

        lax.fori_loop(0, CB, block0, 0)
        plsc.subcore_barrier()

        flush_acc(0)

        @pl.when(s == 0)
        def _():
            pltpu.sync_copy(rs_sh, rs_out.at[c])
            pltpu.sync_copy(rcnt_sh, rcnt_out.at[c])

        zero_acc()
        plsc.subcore_barrier()

        def block1(b, carry):
            pltpu.sync_copy(src_hbm.at[wid, b], src_v)
            pltpu.sync_copy(d1_hbm.at[wid, b], dst_v)
            pltpu.async_copy(h_hbm.at[src_v.at[0]], rows_a, sem_a)

            def scat(rows, j):
                pltpu.sync_copy(rows, a_sh.at[dst_v.at[j]], add=True)
                pltpu.sync_copy(ones_v, deg_sh.at[dst_v.at[j]], add=True)

            def chunk(j, carry2):
                @pl.when(j % 2 == 0)
                def _():
                    pltpu.make_async_copy(h_hbm.at[src_v.at[j]], rows_a,
                                          sem_a).wait()

                    @pl.when(j < _CHB - 1)
                    def _():
                        pltpu.async_copy(h_hbm.at[src_v.at[j + 1]],
                                         rows_b, sem_b)

                    scat(rows_a, j)

                @pl.when(j % 2 == 1)
                def _():
                    pltpu.make_async_copy(h_hbm.at[src_v.at[j]], rows_b,
                                          sem_b).wait()

                    @pl.when(j < _CHB - 1)
                    def _():
                        pltpu.async_copy(h_hbm.at[src_v.at[j + 1]],
                                         rows_a, sem_a)

                    scat(rows_b, j)
                return carry2

            lax.fori_loop(0, _CHB, chunk, 0)
            return carry

        lax.fori_loop(0, CB, block1, 0)
        plsc.subcore_barrier()
        flush_acc(1)

    return pl.kernel(body, out_type=out_type, mesh=mesh,
                     scratch_types=scratch)


def _build_pass2(N, H, R2, E):
    K = _K
    CB = E // (_NW * K * _CHB)
    HN = N // 2
    STR = (HN // _NS) // 8 * 8
    TAIL = HN - STR * _NS
    mesh = plsc.VectorSubcoreMesh(core_axis_name="c", subcore_axis_name="s")
    out_type = jax.ShapeDtypeStruct((_NC, 2, HN, H), jnp.float32)
    scratch = [
        pltpu.VMEM((_CHB, K), jnp.int32),
        pltpu.VMEM((_CHB, K), jnp.int32),
        pltpu.VMEM((K, H), jnp.float32),
        pltpu.VMEM((K, H), jnp.float32),
        pltpu.VMEM_SHARED((HN + 8, H), jnp.float32),
        pltpu.SemaphoreType.DMA,
        pltpu.SemaphoreType.DMA,
    ]

    def body(d0_hbm, d1_hbm, et_hbm, r_hbm, z2_hbm,
             b_out, dst_v, et_v, rows_a, rows_b, b_sh, sem_a, sem_b):
        c = lax.axis_index("c")
        s = lax.axis_index("s")
        wid = c * _NS + s

        def phase(d_hbm, p):
            pltpu.sync_copy(z2_hbm.at[pl.ds(s * STR, STR)],
                            b_sh.at[pl.ds(s * STR, STR)])

            @pl.when(s == _NS - 1)
            def _():
                pltpu.sync_copy(z2_hbm.at[pl.ds(STR * _NS, TAIL)],
                                b_sh.at[pl.ds(STR * _NS, TAIL)])

            plsc.subcore_barrier()

            def block(b, carry):
                pltpu.sync_copy(d_hbm.at[wid, b], dst_v)
                pltpu.sync_copy(et_hbm.at[wid, b], et_v)
                pltpu.async_copy(r_hbm.at[et_v.at[0]], rows_a, sem_a)

                def chunk(j, carry2):
                    @pl.when(j % 2 == 0)
                    def _():
                        pltpu.make_async_copy(r_hbm.at[et_v.at[j]],
                                              rows_a, sem_a).wait()

                        @pl.when(j < _CHB - 1)
                        def _():
                            pltpu.async_copy(r_hbm.at[et_v.at[j + 1]],
                                             rows_b, sem_b)

                        pltpu.sync_copy(rows_a, b_sh.at[dst_v.at[j]],
                                        add=True)

                    @pl.when(j % 2 == 1)
                    def _():
                        pltpu.make_async_copy(r_hbm.at[et_v.at[j]],
                                              rows_b, sem_b).wait()

                        @pl.when(j < _CHB - 1)
                        def _():
                            pltpu.async_copy(r_hbm.at[et_v.at[j + 1]],
                                             rows_a, sem_a)

                        pltpu.sync_copy(rows_b, b_sh.at[dst_v.at[j]],
                                        add=True)
                    return carry2

                lax.fori_loop(0, _CHB, chunk, 0)
                return carry

            lax.fori_loop(0, CB, block, 0)
            plsc.subcore_barrier()
            pltpu.sync_copy(b_sh.at[pl.ds(s * STR, STR)],
                            b_out.at[c, p, pl.ds(s * STR, STR)])

            @pl.when(s == _NS - 1)
            def _():
                pltpu.sync_copy(b_sh.at[pl.ds(STR * _NS, TAIL)],
                                b_out.at[c, p, pl.ds(STR * _NS, TAIL)])

        phase(d0_hbm, 0)
        plsc.subcore_barrier()
        phase(d1_hbm, 1)

    return pl.kernel(body, out_type=out_type, mesh=mesh,
                     scratch_types=scratch)


def _prep(demb):
    N, H = demb.shape
    RB = 2000

    def body(d_ref, h_ref):
        h_ref[...] = _l2n(d_ref[...])

    return pl.pallas_call(
        body,
        grid=(N // RB,),
        in_specs=[pl.BlockSpec((RB, H), lambda i: (i, 0))],
        out_specs=pl.BlockSpec((RB, H), lambda i: (i, 0)),
        out_shape=jax.ShapeDtypeStruct((N, H), jnp.float32),
    )(demb)


def _tc_rel(emb_rel, r_prev, rs_p, rcnt_p, wih, whh, bih, bhh):
    R2, H = emb_rel.shape

    def body(emb_ref, r_ref, rs_ref, rcnt_ref, wih_ref, whh_ref,
             bih_ref, bhh_ref, rout_ref):
        rs = rs_ref[0] + rs_ref[1]
        cnt = jnp.maximum(jnp.sum(rcnt_ref[...], axis=0), 1.0)
        x = jnp.concatenate([emb_ref[...], rs / cnt], axis=1)
        rout_ref[...] = _gru(x, r_ref[...], wih_ref[...], whh_ref[...],
                             bih_ref[...], bhh_ref[...], H)

    return pl.pallas_call(
        body,
        out_shape=jax.ShapeDtypeStruct((R2, H), jnp.float32),
    )(emb_rel, r_prev, rs_p, rcnt_p, wih, whh, bih, bhh)


def _tc_ent(a_p, b_p, deg_p, h, wn, wl, wih, whh, bih, bhh):
    N, H = h.shape
    RB = 2000

    def body(a_ref, b_ref, deg_ref, h_ref, wn_ref, wl_ref, wih_ref,
             whh_ref, bih_ref, bhh_ref, hout_ref):
        acc = a_ref[0] + a_ref[1] - b_ref[0] - b_ref[1]
        deg = jnp.maximum(jnp.sum(deg_ref[...], axis=0), 1.0)
        agg = jnp.dot(acc, wn_ref[...],
                      preferred_element_type=jnp.float32) / deg
        cur = agg + jnp.dot(h_ref[...], wl_ref[...],
                            preferred_element_type=jnp.float32)
        cur = jnp.where(cur >= 0, cur, _SLOPE * cur)
        cur = _l2n(cur)
        hn = _gru(cur, h_ref[...], wih_ref[...], whh_ref[...],
                  bih_ref[...], bhh_ref[...], H)
        hout_ref[...] = _l2n(hn)

    c0 = lambda i: (0, i, 0)
    return pl.pallas_call(
        body,
        grid=(N // RB,),
        in_specs=[pl.BlockSpec((2, RB, H), c0),
                  pl.BlockSpec((2, RB, H), c0),
                  pl.BlockSpec((2, RB, 1), c0),
                  pl.BlockSpec((RB, H), lambda i: (i, 0)),
                  pl.BlockSpec((H, H), lambda i: (0, 0)),
                  pl.BlockSpec((H, H), lambda i: (0, 0)),
                  pl.BlockSpec((3 * H, H), lambda i: (0, 0)),
                  pl.BlockSpec((3 * H, H), lambda i: (0, 0)),
                  pl.BlockSpec((1, 3 * H), lambda i: (0, 0)),
                  pl.BlockSpec((1, 3 * H), lambda i: (0, 0))],
        out_specs=pl.BlockSpec((RB, H), lambda i: (i, 0)),
        out_shape=jax.ShapeDtypeStruct((N, H), jnp.float32),
    )(a_p, b_p, deg_p, h, wn, wl, wih, whh, bih, bhh)


def kernel(edge_index, edge_type, dynamic_emb, emb_rel, W_neighbor, W_loop,
           Wih_r, Whh_r, bih_r, bhh_r, Wih_e, Whh_e, bih_e, bhh_e):
    N, H = dynamic_emb.shape
    R2 = emb_rel.shape[0]
    T, E = edge_type.shape
    f32 = jnp.float32
    CB = E // (_NW * _K * _CHB)
    HN = N // 2

    z2 = jnp.zeros((HN, H), f32)
    z2r = jnp.zeros((R2, H), f32)
    z1 = jnp.zeros((HN + 8,), f32)
    z1r = jnp.zeros((R2,), f32)
    one = jnp.ones((_K,), f32)
    bih_r2 = bih_r.reshape(1, -1)
    bhh_r2 = bhh_r.reshape(1, -1)
    bih_e2 = bih_e.reshape(1, -1)
    bhh_e2 = bhh_e.reshape(1, -1)

    pass1 = _build_pass1(N, H, R2, E)
    pass2 = _build_pass2(N, H, R2, E)

    h = _prep(dynamic_emb)
    r = emb_rel
    for t in range(T):
        src = edge_index[t, 0].astype(jnp.int32).reshape(_NW, CB, _CHB, _K)
        dst = edge_index[t, 1].astype(jnp.int32)
        d0 = jnp.where(dst < HN, dst, HN).reshape(_NW, CB, _CHB, _K)
        d1 = jnp.where(dst >= HN, dst - HN, HN).reshape(_NW, CB, _CHB, _K)
        et = edge_type[t].astype(jnp.int32).reshape(_NW, CB, _CHB, _K)
        a_p, rs_p, deg_p, rcnt_p = pass1(src, d0, d1, et, h, z2, z2r, z1,
                                         z1r, one)
        r = _tc_rel(emb_rel, r, rs_p, rcnt_p.reshape(_NC, R2, 1),
                    Wih_r, Whh_r, bih_r2, bhh_r2)
        b_p = pass2(d0, d1, et, r, z2)
        h = _tc_ent(a_p.reshape(_NC, N, H), b_p.reshape(_NC, N, H),
                    deg_p[:, :, :HN].reshape(_NC, N, 1), h, W_neighbor,
                    W_loop, Wih_e, Whh_e, bih_e2, bhh_e2)
    return h
